# baseline (speedup 1.0000x reference)
import functools

import jax
import jax.numpy as jnp
from jax import lax
from jax.experimental import pallas as pl
from jax.experimental.pallas import tpu as pltpu
from jax.experimental.pallas import tpu_sc as plsc

N_NODES = 10000
N_EDGES = 320000
DIM = 128

NC = 2
NS = 16
NW = NC * NS
K = 128
CHUNKS = 80
E_PAD = NW * CHUNKS * K
N_ACC = 10240
RPT = N_ACC // NS
PUB = RPT // K
CGRP = 8

RB = 2000


def _zero_acc(iota_hbm, pidx_v, zbuf_v, acc_sh, r0):
    for j in range(PUB):
        pltpu.sync_copy(iota_hbm.at[pl.ds(r0 + j * K, K)], pidx_v)
        pltpu.sync_copy(zbuf_v, acc_sh.at[pidx_v])


def _publish_acc(iota_hbm, pidx_v, buf_v, acc_sh, out_hbm, c, r0):
    for j in range(PUB):
        pltpu.sync_copy(iota_hbm.at[pl.ds(r0 + j * K, K)], pidx_v)
        pltpu.sync_copy(acc_sh.at[pidx_v], buf_v)
        pltpu.sync_copy(buf_v, out_hbm.at[c, pl.ds(r0 + j * K, K)])


def _sum_loop(x_hbm, srcg_hbm, dstg_hbm, sidx_v, didx_v, rows_v, acc_sh, sem,
              wid):
    def body(i, carry):
        pltpu.sync_copy(srcg_hbm.at[wid, i], sidx_v)
        pltpu.sync_copy(dstg_hbm.at[wid, i], didx_v)
        pltpu.async_copy(x_hbm.at[sidx_v], rows_v, sem).wait()
        pltpu.sync_copy(rows_v, acc_sh.at[didx_v], add=True)
        return carry

    lax.fori_loop(0, CHUNKS, body, 0)


def _sc_l1_body(x_hbm, srcg_hbm, dstg_hbm, iota_hbm, zrow_hbm,
                ones_hbm, cnt_hbm, part_hbm,
                dtab_v, sidx_v, didx_v, pidx_v, rows_v, ones_v, acc_sh,
                sem, ssem):
    c = lax.axis_index("c")
    s = lax.axis_index("s")
    wid = c * NS + s
    r0 = s * RPT

    idx_load = pltpu.async_copy(dstg_hbm.at[wid], dtab_v, sem)
    pltpu.sync_copy(zrow_hbm, rows_v)
    pltpu.sync_copy(ones_hbm, ones_v)
    _zero_acc(iota_hbm, pidx_v, rows_v, acc_sh, r0)
    idx_load.wait()
    plsc.subcore_barrier()

    def cbody(g, carry):
        b = g * CGRP
        for j in range(CGRP):
            pltpu.async_copy(ones_v, acc_sh.at[dtab_v.at[b + j]], ssem,
                             add=True)
        for j in range(CGRP):
            pltpu.make_async_copy(
                ones_v, acc_sh.at[dtab_v.at[b + j]], ssem).wait()
        return carry

    lax.fori_loop(0, CHUNKS // CGRP, cbody, 0)
    plsc.subcore_barrier()
    _publish_acc(iota_hbm, pidx_v, rows_v, acc_sh, cnt_hbm, c, r0)
    pltpu.sync_copy(zrow_hbm, rows_v)
    _zero_acc(iota_hbm, pidx_v, rows_v, acc_sh, r0)
    plsc.subcore_barrier()

    _sum_loop(x_hbm, srcg_hbm, dstg_hbm, sidx_v, didx_v, rows_v, acc_sh, sem,
              wid)
    plsc.subcore_barrier()
    _publish_acc(iota_hbm, pidx_v, rows_v, acc_sh, part_hbm, c, r0)


def _sc_sum_body(x_hbm, srcg_hbm, dstg_hbm, iota_hbm, zrow_hbm,
                 part_hbm,
                 sidx_v, didx_v, pidx_v, rows_v, acc_sh, sem):
    c = lax.axis_index("c")
    s = lax.axis_index("s")
    wid = c * NS + s
    r0 = s * RPT

    pltpu.sync_copy(zrow_hbm, rows_v)
    _zero_acc(iota_hbm, pidx_v, rows_v, acc_sh, r0)
    plsc.subcore_barrier()
    _sum_loop(x_hbm, srcg_hbm, dstg_hbm, sidx_v, didx_v, rows_v, acc_sh, sem,
              wid)
    plsc.subcore_barrier()
    _publish_acc(iota_hbm, pidx_v, rows_v, acc_sh, part_hbm, c, r0)


@functools.cache
def _make_mesh():
    return plsc.VectorSubcoreMesh(
        core_axis_name="c", subcore_axis_name="s", num_cores=NC,
        num_subcores=NS)


@functools.cache
def _make_sc_l1():
    return pl.kernel(
        _sc_l1_body,
        out_type=(
            jax.ShapeDtypeStruct((NC, N_ACC, DIM), jnp.float32),
            jax.ShapeDtypeStruct((NC, N_ACC, DIM), jnp.float32),
        ),
        mesh=_make_mesh(),
        scratch_types=[
            pltpu.VMEM((CHUNKS, K), jnp.int32),
            pltpu.VMEM((K,), jnp.int32),
            pltpu.VMEM((K,), jnp.int32),
            pltpu.VMEM((K,), jnp.int32),
            pltpu.VMEM((K, DIM), jnp.float32),
            pltpu.VMEM((K, DIM), jnp.float32),
            pltpu.VMEM_SHARED((N_ACC, DIM), jnp.float32),
            pltpu.SemaphoreType.DMA,
            pltpu.SemaphoreType.DMA,
        ],
        name="sage_cnt_sum_sc",
    )


@functools.cache
def _make_sc_sum():
    return pl.kernel(
        _sc_sum_body,
        out_type=jax.ShapeDtypeStruct((NC, N_ACC, DIM), jnp.float32),
        mesh=_make_mesh(),
        scratch_types=[
            pltpu.VMEM((K,), jnp.int32),
            pltpu.VMEM((K,), jnp.int32),
            pltpu.VMEM((K,), jnp.int32),
            pltpu.VMEM((K, DIM), jnp.float32),
            pltpu.VMEM_SHARED((N_ACC, DIM), jnp.float32),
            pltpu.SemaphoreType.DMA,
        ],
        name="sage_segment_sum_sc",
    )


def _tc_layer_kernel(part_ref, cnt_ref, x_ref, wl_ref, wr_ref, b_ref, o_ref):
    cnt = cnt_ref[0, :, 0:1] + cnt_ref[1, :, 0:1]
    recip = 1.0 / jnp.maximum(cnt, 1.0)
    mean = (part_ref[0] + part_ref[1]) * recip
    acc = lax.dot_general(mean, wl_ref[...], (((1,), (1,)), ((), ())),
                          preferred_element_type=jnp.float32)
    acc = acc + lax.dot_general(x_ref[...], wr_ref[...],
                                (((1,), (1,)), ((), ())),
                                preferred_element_type=jnp.float32)
    o_ref[...] = jnp.maximum(acc + b_ref[...], 0.0)


def _tc_layer(part, cnt, x, W_l, W_r, b):
    grid = N_NODES // RB
    return pl.pallas_call(
        _tc_layer_kernel,
        grid=(grid,),
        in_specs=[
            pl.BlockSpec((NC, RB, DIM), lambda i: (0, i, 0)),
            pl.BlockSpec((NC, RB, DIM), lambda i: (0, i, 0)),
            pl.BlockSpec((RB, DIM), lambda i: (i, 0)),
            pl.BlockSpec((DIM, DIM), lambda i: (0, 0)),
            pl.BlockSpec((DIM, DIM), lambda i: (0, 0)),
            pl.BlockSpec((1, DIM), lambda i: (0, 0)),
        ],
        out_specs=pl.BlockSpec((RB, DIM), lambda i: (i, 0)),
        out_shape=jax.ShapeDtypeStruct((N_NODES, DIM), jnp.float32),
        name="sage_dense_tc",
    )(part, cnt, x, W_l, W_r, b.reshape(1, DIM))


def kernel(x, edge_index, W1_l, b1_l, W1_r, W2_l, b2_l, W2_r):
    src = edge_index[0].astype(jnp.int32)
    dst = edge_index[1].astype(jnp.int32)
    pad = E_PAD - N_EDGES
    srcg = jnp.concatenate(
        [src, jnp.zeros((pad,), jnp.int32)]).reshape(NW, CHUNKS, K)
    dstg = jnp.concatenate(
        [dst, jnp.full((pad,), N_NODES, jnp.int32)]).reshape(NW, CHUNKS, K)
    iota = jnp.arange(N_ACC, dtype=jnp.int32)
    zrow = jnp.zeros((K, DIM), jnp.float32)
    ones = jnp.ones((K, DIM), jnp.float32)

    cnt, part1 = _make_sc_l1()(x, srcg, dstg, iota, zrow, ones)
    h1 = _tc_layer(part1, cnt, x, W1_l, W1_r, b1_l)
    part2 = _make_sc_sum()(h1, srcg, dstg, iota, zrow)
    h2 = _tc_layer(part2, cnt, h1, W2_l, W2_r, b2_l)
    return h2


# speedup vs baseline: 1.3488x; 1.3488x over previous
import functools

import jax
import jax.numpy as jnp
from jax import lax
from jax.experimental import pallas as pl
from jax.experimental.pallas import tpu as pltpu
from jax.experimental.pallas import tpu_sc as plsc

N_NODES = 10000
N_EDGES = 320000
DIM = 128

NC = 2
NS = 16
NW = NC * NS
K = 128
CHUNKS = -(-N_EDGES // (NW * K))
E_PAD = NW * CHUNKS * K
N_ACC = 10240
RPT = N_ACC // NS
PUB = RPT // K

CCHUNKS = 80
CE_PAD = NW * CCHUNKS * K
CGRP = 8

RB = 2000


def _sc_sum_body(x_hbm, src_hbm, dst_hbm, iota_hbm, zrow_hbm,
                 part_hbm,
                 sidx_v, didx_v, rows_v, acc_sh, sem):
    c = lax.axis_index("c")
    s = lax.axis_index("s")
    wid = c * NS + s
    r0 = s * RPT
    pltpu.sync_copy(zrow_hbm, rows_v)
    for j in range(PUB):
        pltpu.sync_copy(iota_hbm.at[pl.ds(r0 + j * K, K)], sidx_v)
        pltpu.sync_copy(rows_v, acc_sh.at[sidx_v])
    plsc.subcore_barrier()

    base = wid * (CHUNKS * K)

    def body(i, carry):
        off = base + i * K
        pltpu.sync_copy(src_hbm.at[pl.ds(off, K)], sidx_v)
        pltpu.sync_copy(dst_hbm.at[pl.ds(off, K)], didx_v)
        pltpu.async_copy(x_hbm.at[sidx_v], rows_v, sem).wait()
        pltpu.sync_copy(rows_v, acc_sh.at[didx_v], add=True)
        return carry

    lax.fori_loop(0, CHUNKS, body, 0)
    plsc.subcore_barrier()
    for j in range(PUB):
        pltpu.sync_copy(iota_hbm.at[pl.ds(r0 + j * K, K)], sidx_v)
        pltpu.sync_copy(acc_sh.at[sidx_v], rows_v)
        pltpu.sync_copy(rows_v, part_hbm.at[c, pl.ds(r0 + j * K, K)])


def _sc_cnt_body(dstg_hbm, iota_hbm, zrow_hbm, ones_hbm,
                 cnt_hbm,
                 didx_v, pidx_v, rows_v, ones_v, acc_sh, sem, ssem):
    c = lax.axis_index("c")
    s = lax.axis_index("s")
    wid = c * NS + s
    r0 = s * RPT
    idx_load = pltpu.async_copy(dstg_hbm.at[wid], didx_v, sem)
    pltpu.sync_copy(zrow_hbm, rows_v)
    pltpu.sync_copy(ones_hbm, ones_v)
    for j in range(PUB):
        pltpu.sync_copy(iota_hbm.at[pl.ds(r0 + j * K, K)], pidx_v)
        pltpu.sync_copy(rows_v, acc_sh.at[pidx_v])
    idx_load.wait()
    plsc.subcore_barrier()

    def body(g, carry):
        base = g * CGRP
        for j in range(CGRP):
            pltpu.async_copy(ones_v, acc_sh.at[didx_v.at[base + j]], ssem,
                             add=True)
        for j in range(CGRP):
            pltpu.make_async_copy(
                ones_v, acc_sh.at[didx_v.at[base + j]], ssem).wait()
        return carry

    lax.fori_loop(0, CCHUNKS // CGRP, body, 0)
    plsc.subcore_barrier()
    for j in range(PUB):
        pltpu.sync_copy(iota_hbm.at[pl.ds(r0 + j * K, K)], pidx_v)
        pltpu.sync_copy(acc_sh.at[pidx_v], rows_v)
        pltpu.sync_copy(rows_v, cnt_hbm.at[c, pl.ds(r0 + j * K, K)])


@functools.cache
def _make_mesh():
    return plsc.VectorSubcoreMesh(
        core_axis_name="c", subcore_axis_name="s", num_cores=NC,
        num_subcores=NS)


@functools.cache
def _make_sc_sum():
    return pl.kernel(
        _sc_sum_body,
        out_type=jax.ShapeDtypeStruct((NC, N_ACC, DIM), jnp.float32),
        mesh=_make_mesh(),
        scratch_types=[
            pltpu.VMEM((K,), jnp.int32),
            pltpu.VMEM((K,), jnp.int32),
            pltpu.VMEM((K, DIM), jnp.float32),
            pltpu.VMEM_SHARED((N_ACC, DIM), jnp.float32),
            pltpu.SemaphoreType.DMA,
        ],
        name="sage_segment_sum_sc",
    )


@functools.cache
def _make_sc_cnt():
    return pl.kernel(
        _sc_cnt_body,
        out_type=jax.ShapeDtypeStruct((NC, N_ACC, DIM), jnp.float32),
        mesh=_make_mesh(),
        scratch_types=[
            pltpu.VMEM((CCHUNKS, K), jnp.int32),
            pltpu.VMEM((K,), jnp.int32),
            pltpu.VMEM((K, DIM), jnp.float32),
            pltpu.VMEM((K, DIM), jnp.float32),
            pltpu.VMEM_SHARED((N_ACC, DIM), jnp.float32),
            pltpu.SemaphoreType.DMA,
            pltpu.SemaphoreType.DMA,
        ],
        name="sage_degree_count_sc",
    )


def _tc_layer_kernel(part_ref, cnt_ref, x_ref, wl_ref, wr_ref, b_ref, o_ref):
    cnt = cnt_ref[0, :, 0:1] + cnt_ref[1, :, 0:1]
    recip = 1.0 / jnp.maximum(cnt, 1.0)
    mean = (part_ref[0] + part_ref[1]) * recip
    acc = lax.dot_general(mean, wl_ref[...], (((1,), (1,)), ((), ())),
                          preferred_element_type=jnp.float32)
    acc = acc + lax.dot_general(x_ref[...], wr_ref[...],
                                (((1,), (1,)), ((), ())),
                                preferred_element_type=jnp.float32)
    o_ref[...] = jnp.maximum(acc + b_ref[...], 0.0)


def _tc_layer(part, cnt, x, W_l, W_r, b):
    grid = N_NODES // RB
    return pl.pallas_call(
        _tc_layer_kernel,
        grid=(grid,),
        in_specs=[
            pl.BlockSpec((NC, RB, DIM), lambda i: (0, i, 0)),
            pl.BlockSpec((NC, RB, DIM), lambda i: (0, i, 0)),
            pl.BlockSpec((RB, DIM), lambda i: (i, 0)),
            pl.BlockSpec((DIM, DIM), lambda i: (0, 0)),
            pl.BlockSpec((DIM, DIM), lambda i: (0, 0)),
            pl.BlockSpec((1, DIM), lambda i: (0, 0)),
        ],
        out_specs=pl.BlockSpec((RB, DIM), lambda i: (i, 0)),
        out_shape=jax.ShapeDtypeStruct((N_NODES, DIM), jnp.float32),
        name="sage_dense_tc",
    )(part, cnt, x, W_l, W_r, b.reshape(1, DIM))


def kernel(x, edge_index, W1_l, b1_l, W1_r, W2_l, b2_l, W2_r):
    src = edge_index[0].astype(jnp.int32)
    dst = edge_index[1].astype(jnp.int32)
    srcf = jnp.concatenate(
        [src, jnp.zeros((E_PAD - N_EDGES,), jnp.int32)])
    dstf = jnp.concatenate(
        [dst, jnp.full((E_PAD - N_EDGES,), N_NODES, jnp.int32)])
    dstg = jnp.concatenate(
        [dst, jnp.full((CE_PAD - N_EDGES,), N_NODES, jnp.int32)]
    ).reshape(NW, CCHUNKS, K)
    iota = jnp.arange(N_ACC, dtype=jnp.int32)
    zrow = jnp.zeros((K, DIM), jnp.float32)
    ones = jnp.ones((K, DIM), jnp.float32)

    cnt = _make_sc_cnt()(dstg, iota, zrow, ones)
    part1 = _make_sc_sum()(x, srcf, dstf, iota, zrow)
    h1 = _tc_layer(part1, cnt, x, W1_l, W1_r, b1_l)
    part2 = _make_sc_sum()(h1, srcf, dstf, iota, zrow)
    h2 = _tc_layer(part2, cnt, h1, W2_l, W2_r, b2_l)
    return h2


# speedup vs baseline: 1.4490x; 1.0743x over previous
import functools

import jax
import jax.numpy as jnp
from jax import lax
from jax.experimental import pallas as pl
from jax.experimental.pallas import tpu as pltpu
from jax.experimental.pallas import tpu_sc as plsc

N_NODES = 10000
N_EDGES = 320000
DIM = 128

NC = 2
NS = 16
NW = NC * NS
K = 128
CHUNKS = -(-N_EDGES // (NW * K))
E_PAD = NW * CHUNKS * K
N_ACC = 10240
RPT = N_ACC // NS
PUB = RPT // K

CCHUNKS = 80
CE_PAD = NW * CCHUNKS * K
CGRP = 8

RB = 2000


def _sc_sum_body(x_hbm, src_hbm, dst_hbm, iota_hbm, zrow_hbm,
                 part_hbm,
                 sidx_v, didx_v, rows_v, acc_sh, sem, sem2):
    c = lax.axis_index("c")
    s = lax.axis_index("s")
    wid = c * NS + s
    r0 = s * RPT
    pltpu.sync_copy(zrow_hbm, rows_v)
    for j in range(PUB):
        pltpu.sync_copy(iota_hbm.at[pl.ds(r0 + j * K, K)], sidx_v)
        pltpu.sync_copy(rows_v, acc_sh.at[sidx_v])
    plsc.subcore_barrier()

    base = wid * (CHUNKS * K)

    def body(i, carry):
        off = base + i * K
        pltpu.sync_copy(src_hbm.at[pl.ds(off, K)], sidx_v)
        dload = pltpu.async_copy(dst_hbm.at[pl.ds(off, K)], didx_v, sem2)
        pltpu.async_copy(x_hbm.at[sidx_v], rows_v, sem).wait()
        dload.wait()
        pltpu.sync_copy(rows_v, acc_sh.at[didx_v], add=True)
        return carry

    lax.fori_loop(0, CHUNKS, body, 0)
    plsc.subcore_barrier()
    for j in range(PUB):
        pltpu.sync_copy(iota_hbm.at[pl.ds(r0 + j * K, K)], sidx_v)
        pltpu.sync_copy(acc_sh.at[sidx_v], rows_v)
        pltpu.sync_copy(rows_v, part_hbm.at[c, pl.ds(r0 + j * K, K)])


def _sc_cnt_body(dstg_hbm, iota_hbm, zrow_hbm, ones_hbm,
                 cnt_hbm,
                 didx_v, pidx_v, rows_v, ones_v, acc_sh, sem, ssem):
    c = lax.axis_index("c")
    s = lax.axis_index("s")
    wid = c * NS + s
    r0 = s * RPT
    idx_load = pltpu.async_copy(dstg_hbm.at[wid], didx_v, sem)
    pltpu.sync_copy(zrow_hbm, rows_v)
    pltpu.sync_copy(ones_hbm, ones_v)
    for j in range(PUB):
        pltpu.sync_copy(iota_hbm.at[pl.ds(r0 + j * K, K)], pidx_v)
        pltpu.sync_copy(rows_v, acc_sh.at[pidx_v])
    idx_load.wait()
    plsc.subcore_barrier()

    def body(g, carry):
        base = g * CGRP
        for j in range(CGRP):
            pltpu.async_copy(ones_v, acc_sh.at[didx_v.at[base + j]], ssem,
                             add=True)
        for j in range(CGRP):
            pltpu.make_async_copy(
                ones_v, acc_sh.at[didx_v.at[base + j]], ssem).wait()
        return carry

    lax.fori_loop(0, CCHUNKS // CGRP, body, 0)
    plsc.subcore_barrier()
    for j in range(PUB):
        pltpu.sync_copy(iota_hbm.at[pl.ds(r0 + j * K, K)], pidx_v)
        pltpu.sync_copy(acc_sh.at[pidx_v], rows_v)
        pltpu.sync_copy(rows_v, cnt_hbm.at[c, pl.ds(r0 + j * K, K)])


@functools.cache
def _make_mesh():
    return plsc.VectorSubcoreMesh(
        core_axis_name="c", subcore_axis_name="s", num_cores=NC,
        num_subcores=NS)


@functools.cache
def _make_sc_sum():
    return pl.kernel(
        _sc_sum_body,
        out_type=jax.ShapeDtypeStruct((NC, N_ACC, DIM), jnp.float32),
        mesh=_make_mesh(),
        scratch_types=[
            pltpu.VMEM((K,), jnp.int32),
            pltpu.VMEM((K,), jnp.int32),
            pltpu.VMEM((K, DIM), jnp.float32),
            pltpu.VMEM_SHARED((N_ACC, DIM), jnp.float32),
            pltpu.SemaphoreType.DMA,
            pltpu.SemaphoreType.DMA,
        ],
        name="sage_segment_sum_sc",
    )


@functools.cache
def _make_sc_cnt():
    return pl.kernel(
        _sc_cnt_body,
        out_type=jax.ShapeDtypeStruct((NC, N_ACC, DIM), jnp.float32),
        mesh=_make_mesh(),
        scratch_types=[
            pltpu.VMEM((CCHUNKS, K), jnp.int32),
            pltpu.VMEM((K,), jnp.int32),
            pltpu.VMEM((K, DIM), jnp.float32),
            pltpu.VMEM((K, DIM), jnp.float32),
            pltpu.VMEM_SHARED((N_ACC, DIM), jnp.float32),
            pltpu.SemaphoreType.DMA,
            pltpu.SemaphoreType.DMA,
        ],
        name="sage_degree_count_sc",
    )


def _tc_layer_kernel(part_ref, cnt_ref, x_ref, wl_ref, wr_ref, b_ref, o_ref):
    cnt = cnt_ref[0, :, 0:1] + cnt_ref[1, :, 0:1]
    recip = 1.0 / jnp.maximum(cnt, 1.0)
    mean = (part_ref[0] + part_ref[1]) * recip
    acc = lax.dot_general(mean, wl_ref[...], (((1,), (1,)), ((), ())),
                          preferred_element_type=jnp.float32)
    acc = acc + lax.dot_general(x_ref[...], wr_ref[...],
                                (((1,), (1,)), ((), ())),
                                preferred_element_type=jnp.float32)
    o_ref[...] = jnp.maximum(acc + b_ref[...], 0.0)


def _tc_layer(part, cnt, x, W_l, W_r, b):
    grid = N_NODES // RB
    return pl.pallas_call(
        _tc_layer_kernel,
        grid=(grid,),
        in_specs=[
            pl.BlockSpec((NC, RB, DIM), lambda i: (0, i, 0)),
            pl.BlockSpec((NC, RB, DIM), lambda i: (0, i, 0)),
            pl.BlockSpec((RB, DIM), lambda i: (i, 0)),
            pl.BlockSpec((DIM, DIM), lambda i: (0, 0)),
            pl.BlockSpec((DIM, DIM), lambda i: (0, 0)),
            pl.BlockSpec((1, DIM), lambda i: (0, 0)),
        ],
        out_specs=pl.BlockSpec((RB, DIM), lambda i: (i, 0)),
        out_shape=jax.ShapeDtypeStruct((N_NODES, DIM), jnp.float32),
        name="sage_dense_tc",
    )(part, cnt, x, W_l, W_r, b.reshape(1, DIM))


def kernel(x, edge_index, W1_l, b1_l, W1_r, W2_l, b2_l, W2_r):
    src = edge_index[0].astype(jnp.int32)
    dst = edge_index[1].astype(jnp.int32)
    srcf = jnp.concatenate(
        [src, jnp.zeros((E_PAD - N_EDGES,), jnp.int32)])
    dstf = jnp.concatenate(
        [dst, jnp.full((E_PAD - N_EDGES,), N_NODES, jnp.int32)])
    dstg = jnp.concatenate(
        [dst, jnp.full((CE_PAD - N_EDGES,), N_NODES, jnp.int32)]
    ).reshape(NW, CCHUNKS, K)
    iota = jnp.arange(N_ACC, dtype=jnp.int32)
    zrow = jnp.zeros((K, DIM), jnp.float32)
    ones = jnp.ones((K, DIM), jnp.float32)

    cnt = _make_sc_cnt()(dstg, iota, zrow, ones)
    part1 = _make_sc_sum()(x, srcf, dstf, iota, zrow)
    h1 = _tc_layer(part1, cnt, x, W1_l, W1_r, b1_l)
    part2 = _make_sc_sum()(h1, srcf, dstf, iota, zrow)
    h2 = _tc_layer(part2, cnt, h1, W2_l, W2_r, b2_l)
    return h2
